# baseline (speedup 1.0000x reference)
import functools

import jax
import jax.numpy as jnp
from jax import lax
from jax.experimental import pallas as pl
from jax.experimental.pallas import tpu as pltpu
from jax.experimental.pallas import tpu_sc as plsc

_D = 1024
_MAX_LEN = 32768
_SEQ = 8192
_NC = 2
_NS = 16
_NW = _NC * _NS
_B_PER_W = _SEQ // _NW
_C = 16
_NCHUNK = _B_PER_W // _C
_NBT = 3
_NBS = 3
_G_AHEAD = 2

_mesh = plsc.VectorSubcoreMesh(core_axis_name="c", subcore_axis_name="s")


@functools.partial(
    pl.kernel,
    mesh=_mesh,
    out_type=jax.ShapeDtypeStruct((_SEQ, _D), jnp.float32),
    scratch_types=(
        [pltpu.VMEM((_NCHUNK, _C), jnp.int32),
         pltpu.VMEM((_NBT, _C, _D), jnp.float32),
         pltpu.VMEM_SHARED((_NS, _NBS, _C, _D), jnp.float32)]
        + [pltpu.SemaphoreType.DMA] * (_NBT + _NBS)
    ),
)
def _gather(table_hbm, idx_hbm, out_hbm, idx_v, bufs, spr, *sems):
    cid = lax.axis_index("c")
    sid = lax.axis_index("s")
    wid = sid * _NC + cid
    base = wid * _B_PER_W
    gsem = sems[:_NBT]
    dsem = sems[_NBT:]
    pltpu.sync_copy(idx_hbm.at[wid], idx_v)

    def start_gather(c):
        b = c % _NBT
        return pltpu.async_copy(
            table_hbm.at[idx_v.at[c]], bufs.at[b], gsem[b]
        )

    def start_dma(c):
        b = c % _NBS
        return pltpu.async_copy(
            spr.at[sid, b], out_hbm.at[pl.ds(base + c * _C, _C)], dsem[b]
        )

    gathers = [None] * _NCHUNK
    dmas = [None] * _NCHUNK
    for c in range(_G_AHEAD):
        gathers[c] = start_gather(c)
    for c in range(_NCHUNK):
        if c + _G_AHEAD < _NCHUNK:
            gathers[c + _G_AHEAD] = start_gather(c + _G_AHEAD)
        gathers[c].wait()
        if c >= _NBS:
            dmas[c - _NBS].wait()
        pltpu.sync_copy(bufs.at[c % _NBT], spr.at[sid, c % _NBS])
        dmas[c] = start_dma(c)
    for c in range(_NCHUNK - _NBS, _NCHUNK):
        dmas[c].wait()


def kernel(position_ids, PosEnc):
    table = PosEnc.reshape(_MAX_LEN, _D)
    idx = position_ids.astype(jnp.int32).reshape(_NW, _NCHUNK, _C)
    out = _gather(table, idx)
    return out.reshape(1, _SEQ, _D)


# speedup vs baseline: 1.0259x; 1.0259x over previous
import functools

import jax
import jax.numpy as jnp
from jax import lax
from jax.experimental import pallas as pl
from jax.experimental.pallas import tpu as pltpu
from jax.experimental.pallas import tpu_sc as plsc

_D = 1024
_MAX_LEN = 32768
_SEQ = 8192
_NC = 2
_NS = 16
_NW = _NC * _NS
_B_PER_W = _SEQ // _NW
_C = 32
_NCHUNK = _B_PER_W // _C
_NBUF = 3
_G_AHEAD = 2
_S_OUT = _NBUF - _G_AHEAD

_mesh = plsc.VectorSubcoreMesh(core_axis_name="c", subcore_axis_name="s")


@functools.partial(
    pl.kernel,
    mesh=_mesh,
    out_type=jax.ShapeDtypeStruct((_SEQ, _D), jnp.float32),
    scratch_types=(
        [pltpu.VMEM((_B_PER_W,), jnp.int32),
         pltpu.VMEM((_NBUF, _C, _D), jnp.float32)]
        + [pltpu.SemaphoreType.DMA] * (2 * _NBUF)
    ),
)
def _gather(table_hbm, idx_hbm, out_hbm, idx_v, bufs, *sems):
    cid = lax.axis_index("c")
    sid = lax.axis_index("s")
    wid = sid * _NC + cid
    base = wid * _B_PER_W
    gsem = sems[:_NBUF]
    ssem = sems[_NBUF:]
    pltpu.sync_copy(idx_hbm.at[pl.ds(base, _B_PER_W)], idx_v)

    def start_gather(c):
        b = c % _NBUF
        return pltpu.async_copy(
            table_hbm.at[idx_v.at[pl.ds(c * _C, _C)]], bufs.at[b], gsem[b]
        )

    def start_store(c):
        b = c % _NBUF
        return pltpu.async_copy(
            bufs.at[b], out_hbm.at[pl.ds(base + c * _C, _C)], ssem[b]
        )

    gathers = [None] * _NCHUNK
    stores = [None] * _NCHUNK
    for c in range(_G_AHEAD):
        gathers[c] = start_gather(c)
    for c in range(_NCHUNK):
        if c >= _S_OUT:
            stores[c - _S_OUT].wait()
        if c + _G_AHEAD < _NCHUNK:
            gathers[c + _G_AHEAD] = start_gather(c + _G_AHEAD)
        gathers[c].wait()
        stores[c] = start_store(c)
    for c in range(_NCHUNK - _S_OUT, _NCHUNK):
        stores[c].wait()


def kernel(position_ids, PosEnc):
    table = PosEnc.reshape(_MAX_LEN, _D)
    idx = position_ids.astype(jnp.int32)
    out = _gather(table, idx)
    return out.reshape(1, _SEQ, _D)


# speedup vs baseline: 1.0399x; 1.0136x over previous
import functools

import jax
import jax.numpy as jnp
from jax import lax
from jax.experimental import pallas as pl
from jax.experimental.pallas import tpu as pltpu
from jax.experimental.pallas import tpu_sc as plsc

_D = 1024
_MAX_LEN = 32768
_SEQ = 8192
_NC = 2
_NS = 16
_NW = _NC * _NS
_B_PER_W = _SEQ // _NW
_C = 16
_NCHUNK = _B_PER_W // _C
_NBUF = 6
_G_AHEAD = 4
_S_OUT = _NBUF - _G_AHEAD

_mesh = plsc.VectorSubcoreMesh(core_axis_name="c", subcore_axis_name="s")


@functools.partial(
    pl.kernel,
    mesh=_mesh,
    out_type=jax.ShapeDtypeStruct((_SEQ, _D), jnp.float32),
    scratch_types=(
        [pltpu.VMEM((_B_PER_W,), jnp.int32),
         pltpu.VMEM((_NBUF, _C, _D), jnp.float32)]
        + [pltpu.SemaphoreType.DMA] * (2 * _NBUF)
    ),
)
def _gather(table_hbm, idx_hbm, out_hbm, idx_v, bufs, *sems):
    cid = lax.axis_index("c")
    sid = lax.axis_index("s")
    wid = sid * _NC + cid
    base = wid * _B_PER_W
    gsem = sems[:_NBUF]
    ssem = sems[_NBUF:]
    pltpu.sync_copy(idx_hbm.at[pl.ds(base, _B_PER_W)], idx_v)

    def start_gather(c):
        b = c % _NBUF
        return pltpu.async_copy(
            table_hbm.at[idx_v.at[pl.ds(c * _C, _C)]], bufs.at[b], gsem[b]
        )

    def start_store(c):
        b = c % _NBUF
        return pltpu.async_copy(
            bufs.at[b], out_hbm.at[pl.ds(base + c * _C, _C)], ssem[b]
        )

    gathers = [None] * _NCHUNK
    stores = [None] * _NCHUNK
    for c in range(_G_AHEAD):
        gathers[c] = start_gather(c)
    for c in range(_NCHUNK):
        if c >= _S_OUT:
            stores[c - _S_OUT].wait()
        if c + _G_AHEAD < _NCHUNK:
            gathers[c + _G_AHEAD] = start_gather(c + _G_AHEAD)
        gathers[c].wait()
        stores[c] = start_store(c)
    for c in range(_NCHUNK - _S_OUT, _NCHUNK):
        stores[c].wait()


def kernel(position_ids, PosEnc):
    table = PosEnc.reshape(_MAX_LEN, _D)
    idx = position_ids.astype(jnp.int32)
    out = _gather(table, idx)
    return out.reshape(1, _SEQ, _D)
